# baseline (speedup 1.0000x reference)
import jax
import jax.numpy as jnp
from jax import lax
from jax.experimental import pallas as pl

_N = 16
_H = 4
_OMEGA0 = 30.0
_CIN = 64
_COUT = 64
_A = 16.0

_INV_PI = 0.31830988618367906
_PI_HI = 3.140625
_PI_MID = 9.676536e-4
_PI_LO = 5.126566e-12
_S0 = 1.0
_S1 = -0.1666666
_S2 = 0.008333097
_S3 = -0.00019812485
_S4 = 2.6129003e-06


def _fast_sin(z):
    kf = jnp.round(z * _INV_PI)
    k = kf.astype(jnp.int32)
    r = z - kf * _PI_HI
    r = r - kf * _PI_MID
    s = r * r
    p = _S4
    for c in (_S3, _S2, _S1, _S0):
        p = p * s + c
    p = p * r
    return jnp.where((k & 1) == 1, -p, p)


def _moe_block_kernel(x_ref, c_ref, w_ref, e_ref, s_ref, o_ref):
    xb = x_ref[...]
    cb = c_ref[...]
    wcat = w_ref[...]
    exp_cols = e_ref[...]
    sel = s_ref[...]

    affine = cb * _A
    xi = jnp.floor(affine[:, 0:1]).astype(jnp.int32) % _H
    yi = jnp.floor(affine[:, 1:2]).astype(jnp.int32) % _H
    tid = _H * xi + yi

    y = jnp.dot(xb, wcat, preferred_element_type=jnp.float32)

    ids = lax.broadcasted_iota(jnp.int32, (1, _N), 1)
    onehot = jnp.where(tid == ids, 1.0, 0.0).astype(jnp.bfloat16)
    maskf = jnp.dot(onehot, exp_cols,
                    preferred_element_type=jnp.float32)
    z = y * maskf
    acc128 = z[:, 0:128]
    for g in range(1, _N // 2):
        acc128 = acc128 + z[:, g * 128:(g + 1) * 128]
    acc = jnp.dot(acc128, sel, preferred_element_type=jnp.float32)
    o_ref[...] = _fast_sin(_OMEGA0 * acc)


@jax.jit
def kernel(in_feats, in_coords, W):
    B = in_feats.shape[0]
    bt = 4096
    coords = in_coords.reshape(B, 2)
    wcat = jnp.transpose(W.reshape(_N * _COUT, _CIN))
    sel = jnp.tile(jnp.eye(_COUT, dtype=jnp.float32), (2, 1))
    exp_cols = jnp.repeat(jnp.eye(_N, dtype=jnp.bfloat16), _COUT, axis=1)

    out = pl.pallas_call(
        _moe_block_kernel,
        grid=(B // bt,),
        in_specs=[
            pl.BlockSpec((bt, _CIN), lambda i: (i, 0)),
            pl.BlockSpec((bt, 2), lambda i: (i, 0)),
            pl.BlockSpec((_CIN, _N * _COUT), lambda i: (0, 0)),
            pl.BlockSpec((_N, _N * _COUT), lambda i: (0, 0)),
            pl.BlockSpec((2 * _COUT, _COUT), lambda i: (0, 0)),
        ],
        out_specs=pl.BlockSpec((bt, _COUT), lambda i: (i, 0)),
        out_shape=jax.ShapeDtypeStruct((B, _COUT), jnp.float32),
    )(in_feats, coords, wcat, exp_cols, sel)
    return out


# speedup vs baseline: 1.1675x; 1.1675x over previous
import jax
import jax.numpy as jnp
from jax import lax
from jax.experimental import pallas as pl

_N = 16
_H = 4
_OMEGA0 = 30.0
_CIN = 64
_COUT = 64
_A = 16.0

_INV_PI = 0.31830988618367906
_PI_HI = 3.140625
_PI_MID = 9.676536e-4
_PI_LO = 5.126566e-12
_S0 = 1.0
_S0 = 0.99999946
_S1 = -0.16665891
_S2 = 0.008315963
_S3 = -0.0001860891


def _fast_sin(z):
    kf = jnp.round(z * _INV_PI)
    k = kf.astype(jnp.int32)
    r = z - kf * _PI_HI
    r = r - kf * _PI_MID
    s = r * r
    p = _S3
    for c in (_S2, _S1, _S0):
        p = p * s + c
    p = p * r
    return jnp.where((k & 1) == 1, -p, p)


def _moe_block_kernel(x_ref, c_ref, w_ref, s_ref, o_ref):
    xb = x_ref[...]
    cb = c_ref[...]
    wcat = w_ref[...]
    sel = s_ref[...]

    affine = cb * _A
    xi = jnp.floor(affine[:, 0]).astype(jnp.int32) % _H
    yi = jnp.floor(affine[:, 1]).astype(jnp.int32) % _H
    tid = _H * xi + yi

    y = jnp.dot(xb, wcat, preferred_element_type=jnp.float32)

    bt = xb.shape[0]
    col_expert = lax.broadcasted_iota(jnp.int32, (bt, _N * _COUT), 1) // _COUT
    mask = col_expert == tid[:, None]
    z = jnp.where(mask, y, 0.0)
    acc128 = z[:, 0:128]
    for g in range(1, _N // 2):
        acc128 = acc128 + z[:, g * 128:(g + 1) * 128]
    acc = jnp.dot(acc128, sel, preferred_element_type=jnp.float32)
    o_ref[...] = _fast_sin(acc)


@jax.jit
def kernel(in_feats, in_coords, W):
    B = in_feats.shape[0]
    bt = 4096
    coords = in_coords.reshape(B, 2)
    wcat = jnp.transpose(W.reshape(_N * _COUT, _CIN))
    sel = _OMEGA0 * jnp.tile(jnp.eye(_COUT, dtype=jnp.float32), (2, 1))

    out = pl.pallas_call(
        _moe_block_kernel,
        grid=(B // bt,),
        in_specs=[
            pl.BlockSpec((bt, _CIN), lambda i: (i, 0)),
            pl.BlockSpec((bt, 2), lambda i: (i, 0)),
            pl.BlockSpec((_CIN, _N * _COUT), lambda i: (0, 0)),
            pl.BlockSpec((2 * _COUT, _COUT), lambda i: (0, 0)),
        ],
        out_specs=pl.BlockSpec((bt, _COUT), lambda i: (i, 0)),
        out_shape=jax.ShapeDtypeStruct((B, _COUT), jnp.float32),
    )(in_feats, coords, wcat, sel)
    return out
